# baseline (speedup 1.0000x reference)
import functools

import jax
import jax.numpy as jnp
from jax import lax
from jax.experimental import pallas as pl
from jax.experimental.pallas import tpu as pltpu
from jax.experimental.pallas import tpu_sc as plsc

D = 64
P = 20
L = 16
NC = 2
NS = 16
NW = NC * NS


def _sc_losses(pos_u, pos_v, neg_v, W):
    B = pos_u.shape[0]
    BPW = B // NW
    NCH = BPW // L
    E = L * P
    inv_b = 1.0 / B

    mesh = plsc.VectorSubcoreMesh(
        core_axis_name="c", subcore_axis_name="s", num_cores=NC, num_subcores=NS
    )

    @functools.partial(
        pl.kernel,
        mesh=mesh,
        compiler_params=pltpu.CompilerParams(
            needs_layout_passes=False, use_tc_tiling_on_sc=False
        ),
        out_type=[
            jax.ShapeDtypeStruct((B,), jnp.float32),
            jax.ShapeDtypeStruct((B,), jnp.float32),
        ],
        scratch_types=[
            pltpu.VMEM((BPW,), jnp.int32),
            pltpu.VMEM((128,), jnp.int32),
            pltpu.VMEM((128,), jnp.int32),
            pltpu.VMEM((64,), jnp.int32),
            pltpu.VMEM((128,), jnp.int32),
            pltpu.VMEM((128,), jnp.int32),
            pltpu.VMEM((64,), jnp.int32),
            pltpu.VMEM((BPW, D), jnp.float32),
            pltpu.VMEM((E, D), jnp.float32),
            pltpu.VMEM((E, D), jnp.float32),
            pltpu.VMEM((BPW,), jnp.float32),
            pltpu.VMEM((BPW,), jnp.float32),
            pltpu.SemaphoreType.DMA,
        ],
    )
    def body(pos_u_hbm, pos_v_hbm, neg_v_hbm, w_hbm, pos_out, neg_out,
             idx_u, idx_p0, idx_p1, idx_p2, idx_n0, idx_n1, idx_n2,
             rows_u, rows_p, rows_n, out_pos_v, out_neg_v, sem):
        wid = lax.axis_index("s") * NC + lax.axis_index("c")
        base = wid * BPW

        pltpu.sync_copy(pos_u_hbm.at[pl.ds(base, BPW)], idx_u)
        pltpu.async_copy(w_hbm.at[idx_u], rows_u, sem).wait()

        iota = lax.iota(jnp.int32, L)
        prow = iota * P

        for c in range(NCH):
            eoff = (base + c * L) * P
            pltpu.sync_copy(pos_v_hbm.at[pl.ds(eoff, 128)], idx_p0)
            pltpu.sync_copy(pos_v_hbm.at[pl.ds(eoff + 128, 128)], idx_p1)
            pltpu.sync_copy(pos_v_hbm.at[pl.ds(eoff + 256, 64)], idx_p2)
            pltpu.sync_copy(neg_v_hbm.at[pl.ds(eoff, 128)], idx_n0)
            pltpu.sync_copy(neg_v_hbm.at[pl.ds(eoff + 128, 128)], idx_n1)
            pltpu.sync_copy(neg_v_hbm.at[pl.ds(eoff + 256, 64)], idx_n2)
            cps = [
                pltpu.async_copy(w_hbm.at[idx_p0], rows_p.at[pl.ds(0, 128)], sem),
                pltpu.async_copy(w_hbm.at[idx_p1], rows_p.at[pl.ds(128, 128)], sem),
                pltpu.async_copy(w_hbm.at[idx_p2], rows_p.at[pl.ds(256, 64)], sem),
                pltpu.async_copy(w_hbm.at[idx_n0], rows_n.at[pl.ds(0, 128)], sem),
                pltpu.async_copy(w_hbm.at[idx_n1], rows_n.at[pl.ds(128, 128)], sem),
                pltpu.async_copy(w_hbm.at[idx_n2], rows_n.at[pl.ds(256, 64)], sem),
            ]
            for cp in cps:
                cp.wait()

            urow = c * L + iota

            def dbody(d, carry):
                col = jnp.full((L,), d, dtype=jnp.int32)
                u = plsc.load_gather(rows_u, [urow, col])
                ps = plsc.load_gather(rows_p, [prow, col])
                for p in range(1, P):
                    ps = ps + plsc.load_gather(rows_p, [prow + p, col])
                acc_pos = carry[0] + u * ps
                accs_neg = tuple(
                    carry[1 + n] + u * plsc.load_gather(rows_n, [prow + n, col])
                    for n in range(P)
                )
                return (acc_pos,) + accs_neg

            init = tuple(jnp.zeros((L,), jnp.float32) for _ in range(P + 1))
            res = lax.fori_loop(0, D, dbody, init)

            pos_vec = res[0] * inv_b
            neg_vec = jnp.zeros((L,), jnp.float32)
            for n in range(P):
                s = res[1 + n] * inv_b
                neg_vec = neg_vec + jnp.where(s > 0.0, jnp.exp(s), 0.0)
            out_pos_v[pl.ds(c * L, L)] = pos_vec
            out_neg_v[pl.ds(c * L, L)] = neg_vec

        pltpu.sync_copy(out_pos_v, pos_out.at[pl.ds(base, BPW)])
        pltpu.sync_copy(out_neg_v, neg_out.at[pl.ds(base, BPW)])

    return body(pos_u, pos_v, neg_v, W)


def _combine_body(pos_ref, neg_ref, out_ref):
    out_ref[...] = jnp.log(1.0 + neg_ref[...]) - pos_ref[...]


def kernel(pos_u, pos_v, neg_v, W):
    B = pos_u.shape[0]
    pos_loss, neg_raw = _sc_losses(
        pos_u.reshape(-1).astype(jnp.int32),
        pos_v.reshape(-1).astype(jnp.int32),
        neg_v.reshape(-1).astype(jnp.int32),
        W,
    )
    out = pl.pallas_call(
        _combine_body,
        out_shape=jax.ShapeDtypeStruct((B // 128, 128), jnp.float32),
    )(pos_loss.reshape(B // 128, 128), neg_raw.reshape(B // 128, 128))
    return out.reshape(B)
